# baseline (speedup 1.0000x reference)
import functools

import jax
import jax.numpy as jnp
from jax import lax
from jax.experimental import pallas as pl
from jax.experimental.pallas import tpu as pltpu
from jax.experimental.pallas import tpu_sc as plsc

B = 16
NA = 3
G = 52
C = 80
ROW = C + 5
NCELL = B * NA * G * G
NT = 256
STRIDE = 8.0
LN2 = 0.6931471805599453
AW = (10.0, 16.0, 33.0)
AH = (13.0, 30.0, 23.0)
SAW = tuple(a / STRIDE for a in AW)
SAH = tuple(a / STRIDE for a in AH)

NCH16 = NCELL // 16
NCHUNK = (NCH16 + 31) // 32


def _lnraw(y):
    bits = lax.bitcast_convert_type(y, jnp.int32)
    e = (bits >> 23) - 127
    m = lax.bitcast_convert_type((bits & 0x007FFFFF) | 0x3F800000, jnp.float32)
    big = m > 1.4142135623730951
    m = jnp.where(big, m * 0.5, m)
    e = jnp.where(big, e + 1, e)
    r = (m - 1.0) / (m + 1.0)
    r2 = r * r
    p = r * (2.0 + r2 * (2.0 / 3.0 + r2 * (0.4 + r2 * (2.0 / 7.0 + r2 * (2.0 / 9.0)))))
    return e.astype(jnp.float32) * LN2 + p


def _ln(y):
    return _lnraw(jnp.clip(y, 1e-12, 1.0))


def _sqrt(y):
    return jnp.exp(0.5 * _lnraw(jnp.maximum(y, 1e-36)))


def _chunk_params(tgt_v, off):
    i16 = lax.iota(jnp.int32, 16)
    tb = tgt_v[pl.ds(off, 16)].astype(jnp.int32)
    lab = tgt_v[pl.ds(256 + off, 16)].astype(jnp.int32)
    gx = tgt_v[pl.ds(512 + off, 16)] * float(G)
    gy = tgt_v[pl.ds(768 + off, 16)] * float(G)
    gw = tgt_v[pl.ds(1024 + off, 16)] * float(G)
    gh = tgt_v[pl.ds(1280 + off, 16)] * float(G)
    ious = []
    for a in range(NA):
        inter = jnp.minimum(SAW[a], gw) * jnp.minimum(SAH[a], gh)
        union = SAW[a] * SAH[a] + gw * gh - inter + 1e-16
        ious.append(inter / union)
    zero16 = jnp.zeros((16,), jnp.int32)
    best = zero16
    bv = ious[0]
    best = jnp.where(ious[1] > bv, 1, best)
    bv = jnp.maximum(bv, ious[1])
    best = jnp.where(ious[2] > bv, 2, best)
    gii = jnp.clip(gx.astype(jnp.int32), 0, G - 1)
    gji = jnp.clip(gy.astype(jnp.int32), 0, G - 1)
    cell = ((tb * NA + best) * G + gji) * G + gii
    del i16, zero16
    return dict(tb=tb, lab=lab, gx=gx, gy=gy, gw=gw, gh=gh, ious=ious,
                best=best, gii=gii, gji=gji, cell=cell)


def _body(pred_hbm, tgt_hbm, copy_hbm, out_hbm, tgt_v, cells_v, rows_v,
          rb0_v, rb1_v, rb2_v, rb3_v, lbuf_v, part_v, sem3,
          rs0, rs1, rs2, rs3, ws0, ws1, ws2, ws3):
    i16 = lax.iota(jnp.int32, 16)
    f16 = i16.astype(jnp.float32)
    ones = jnp.ones((16,), jnp.float32)
    zeros = jnp.zeros((16,), jnp.float32)
    sid = lax.axis_index("s")
    cid = lax.axis_index("c")
    wid = sid * 2 + cid
    m = wid & 15

    pltpu.sync_copy(tgt_hbm, tgt_v)

    def p1(k, carry):
        prm = _chunk_params(tgt_v, k * 16)
        cells_v[pl.ds(k * 16, 16)] = prm["cell"]
        return carry

    lax.fori_loop(0, 16, p1, 0)

    prm = _chunk_params(tgt_v, m * 16)
    cell16 = prm["cell"]
    best = prm["best"]
    gx, gy, gw, gh = prm["gx"], prm["gy"], prm["gw"], prm["gh"]
    flrx = gx.astype(jnp.int32).astype(jnp.float32)
    flry = gy.astype(jnp.int32).astype(jnp.float32)
    tx16 = gx - flrx
    ty16 = (gy - flry + 0.5) * 0.5
    aw16 = jnp.where(best == 0, AW[0], jnp.where(best == 1, AW[1], AW[2]))
    ah16 = jnp.where(best == 0, AH[0], jnp.where(best == 1, AH[1], AH[2]))
    tw16 = _sqrt(gw / (aw16 / STRIDE)) * 0.5
    th16 = _sqrt(gh / (ah16 / STRIDE)) * 0.5
    gif = prm["gii"].astype(jnp.float32)
    gjf = prm["gji"].astype(jnp.float32)

    acc = jnp.zeros((16,), jnp.float32)

    @pl.when(wid < 16)
    def _obj():
        t16 = m * 16 + i16
        def dedup(tp, dup):
            cs = plsc.load_gather(cells_v, [jnp.broadcast_to(tp, (16,))])
            hit = (cell16 == cs) & (tp > t16)
            return jnp.where(hit, 1, dup)

        dup = lax.fori_loop(0, NT, dedup, jnp.zeros((16,), jnp.int32))
        wf = 1.0 - dup.astype(jnp.float32)

        for r in range(5):
            def fire(j, carry):
                k = r * 17 + j
                idx = cell16 * ROW + k
                pltpu.async_copy(pred_hbm.at[idx], rows_v.at[pl.ds(k * 16, 16)], sem3)
                return carry

            lax.fori_loop(0, 17, fire, 0)

            def drain(j, carry):
                k = r * 17 + j
                pltpu.make_async_copy(
                    pred_hbm.at[pl.ds(0, 16)], rows_v.at[pl.ds(k * 16, 16)], sem3
                ).wait()
                return carry

            lax.fori_loop(0, 17, drain, 0)

        idxl = cell16 * ROW + 5 + prm["lab"]
        pltpu.async_copy(pred_hbm.at[idxl], lbuf_v, sem3)
        pltpu.make_async_copy(pred_hbm.at[pl.ds(0, 16)], lbuf_v, sem3).wait()

        px = (rows_v[pl.ds(0, 16)] / STRIDE - gif + 0.5) * 0.5
        py = (rows_v[pl.ds(16, 16)] / STRIDE - gjf + 0.5) * 0.5
        pw = _sqrt(rows_v[pl.ds(32, 16)] / aw16) * 0.5
        ph = _sqrt(rows_v[pl.ds(48, 16)] / ah16) * 0.5
        conf = rows_v[pl.ds(64, 16)]
        lx = wf * (px - tx16) * (px - tx16)
        ly = wf * (py - ty16) * (py - ty16)
        lw = wf * (pw - tw16) * (pw - tw16)
        lh = wf * (ph - th16) * (ph - th16)
        cobj = wf * -_ln(conf)

        def clsbody(k, a):
            v = rows_v[pl.ds(k * 16, 16)]
            return a + -_ln(1.0 - v)

        scls = lax.fori_loop(5, ROW, clsbody, jnp.zeros((16,), jnp.float32))
        pl_ = lbuf_v[...]
        clstot = wf * scls + (-_ln(pl_) + _ln(1.0 - pl_))

        part = zeros
        part = jnp.where(f16 == 1.0, jnp.sum(lx), part)
        part = jnp.where(f16 == 2.0, jnp.sum(ly), part)
        part = jnp.where(f16 == 3.0, jnp.sum(lw), part)
        part = jnp.where(f16 == 4.0, jnp.sum(lh), part)
        part = jnp.where(f16 == 5.0, jnp.sum(cobj), part)
        part = jnp.where(f16 == 6.0, jnp.sum(clstot), part)
        part = jnp.where(f16 == 9.0, jnp.sum(wf), part)
        part_v[...] = part

    @pl.when(wid >= 16)
    def _noobj():
        zcells = []
        for a in range(NA):
            zc = ((prm["tb"] * NA + a) * G + prm["gji"]) * G + prm["gii"]
            zcells.append(zc)
            pltpu.async_copy(
                pred_hbm.at[zc * ROW + 4], rows_v.at[pl.ds(a * 16, 16)], sem3
            )
        zs = jnp.zeros((16,), jnp.float32)
        nz = jnp.zeros((16,), jnp.float32)
        for a in range(NA):
            pltpu.make_async_copy(
                pred_hbm.at[pl.ds(0, 16)], rows_v.at[pl.ds(a * 16, 16)], sem3
            ).wait()
            zb = ((best == a) | (prm["ious"][a] > 0.5)).astype(jnp.float32)
            zs = zs + zb * -_ln(1.0 - rows_v[pl.ds(a * 16, 16)])
            nz = nz + zb
        part = zeros
        part = jnp.where(f16 == 7.0, jnp.sum(zs), part)
        part = jnp.where(f16 == 8.0, jnp.sum(nz), part)
        part_v[...] = part

    rbufs = (rb0_v, rb1_v, rb2_v, rb3_v)
    rsems = (rs0, rs1, rs2, rs3)
    wsems = (ws0, ws1, ws2, ws3)
    RLEN = [21760] * 15 + [18360]
    CELLS = [256] * 15 + [216]
    NJ = [16] * 15 + [14]
    ebase = wid * (4056 * ROW)

    def fire_read(d, b):
        n = RLEN[d]
        pltpu.async_copy(
            pred_hbm.at[pl.ds(ebase + d * 21760, n)], rbufs[b].at[pl.ds(0, n)],
            rsems[b])

    def wait_read(d, b):
        n = RLEN[d]
        pltpu.make_async_copy(
            pred_hbm.at[pl.ds(0, n)], rbufs[b].at[pl.ds(0, n)], rsems[b]
        ).wait()

    def fire_write(d, b):
        n = RLEN[d]
        pltpu.async_copy(
            rbufs[b].at[pl.ds(0, n)], copy_hbm.at[pl.ds(ebase + d * 21760, n)],
            wsems[b])

    def wait_write(d, b):
        n = RLEN[d]
        pltpu.make_async_copy(
            rbufs[b].at[pl.ds(0, n)], copy_hbm.at[pl.ds(0, n)], wsems[b]
        ).wait()

    def chunk_sum(d, b, a0):
        full = d < 15

        def jbody(j, a):
            pos = j * 16 + i16
            idx = pos * ROW + 4
            if full:
                return a + -_ln(1.0 - plsc.load_gather(rbufs[b], [idx]))
            w = jnp.where(pos < CELLS[d], ones, zeros)
            idx = jnp.minimum(idx, RLEN[d] - 1)
            return a + w * -_ln(1.0 - plsc.load_gather(rbufs[b], [idx]))

        return lax.fori_loop(0, NJ[d], jbody, a0)

    acc16 = jnp.zeros((16,), jnp.float32)
    fire_read(0, 0)
    fire_read(1, 1)
    for d in range(16):
        b = d & 3
        wait_read(d, b)
        acc16 = chunk_sum(d, b, acc16)
        fire_write(d, b)
        if d + 2 < 16:
            b2 = (d + 2) & 3
            if d - 2 >= 0:
                wait_write(d - 2, b2)
            fire_read(d + 2, b2)
    for d in (12, 13, 14, 15):
        wait_write(d, d & 3)

    part_v[...] = part_v[...] + jnp.where(f16 == 0.0, jnp.sum(acc16), zeros)
    pltpu.sync_copy(part_v, out_hbm.at[wid])


@functools.partial(jax.jit, donate_argnums=())
def _sc_parts(pred_flat, tgt_flat):
    mesh = plsc.VectorSubcoreMesh(
        core_axis_name="c", subcore_axis_name="s", num_cores=2, num_subcores=16
    )
    return pl.kernel(
        _body,
        out_type=(
            jax.ShapeDtypeStruct((NCELL * ROW,), jnp.float32),
            jax.ShapeDtypeStruct((32, 16), jnp.float32),
        ),
        mesh=mesh,
        compiler_params=pltpu.CompilerParams(needs_layout_passes=False),
        scratch_types=[
            pltpu.VMEM((6 * NT,), jnp.float32),
            pltpu.VMEM((NT,), jnp.int32),
            pltpu.VMEM((ROW * 16,), jnp.float32),
            pltpu.VMEM((21760,), jnp.float32),
            pltpu.VMEM((21760,), jnp.float32),
            pltpu.VMEM((21760,), jnp.float32),
            pltpu.VMEM((21760,), jnp.float32),
            pltpu.VMEM((16,), jnp.float32),
            pltpu.VMEM((16,), jnp.float32),
            pltpu.SemaphoreType.DMA,
            pltpu.SemaphoreType.DMA,
            pltpu.SemaphoreType.DMA,
            pltpu.SemaphoreType.DMA,
            pltpu.SemaphoreType.DMA,
            pltpu.SemaphoreType.DMA,
            pltpu.SemaphoreType.DMA,
            pltpu.SemaphoreType.DMA,
            pltpu.SemaphoreType.DMA,
        ],
    )(pred_flat, tgt_flat)


def kernel(pred, targets):
    pred_flat = pred.reshape(-1)
    tgt_flat = jnp.transpose(targets).reshape(-1)
    pred_copy, parts = _sc_parts(pred_flat, tgt_flat)
    s = jnp.sum(parts, axis=0)
    dense, lx, ly, lw, lh, cobj, cls_ = s[0], s[1], s[2], s[3], s[4], s[5], s[6]
    zsum, nzero, nobj = s[7], s[8], s[9]
    total = (
        10.0 * (lx + ly + lw + lh) / nobj
        + cobj / nobj
        + 100.0 * (dense - zsum) / (float(NCELL) - nzero)
        + 10.0 * cls_ / (nobj * float(C))
    )
    return (pred_copy.reshape(pred.shape), total.astype(jnp.float32))


# speedup vs baseline: 1.2454x; 1.2454x over previous
import functools

import jax
import jax.numpy as jnp
from jax import lax
from jax.experimental import pallas as pl
from jax.experimental.pallas import tpu as pltpu
from jax.experimental.pallas import tpu_sc as plsc

B = 16
NA = 3
G = 52
C = 80
ROW = C + 5
NCELL = B * NA * G * G
NT = 256
STRIDE = 8.0
LN2 = 0.6931471805599453
AW = (10.0, 16.0, 33.0)
AH = (13.0, 30.0, 23.0)
SAW = tuple(a / STRIDE for a in AW)
SAH = tuple(a / STRIDE for a in AH)

NCH16 = NCELL // 16
NCHUNK = (NCH16 + 31) // 32


def _lnraw(y):
    bits = lax.bitcast_convert_type(y, jnp.int32)
    e = (bits >> 23) - 127
    m = lax.bitcast_convert_type((bits & 0x007FFFFF) | 0x3F800000, jnp.float32)
    big = m > 1.4142135623730951
    m = jnp.where(big, m * 0.5, m)
    e = jnp.where(big, e + 1, e)
    r = (m - 1.0) / (m + 1.0)
    r2 = r * r
    p = r * (2.0 + r2 * (2.0 / 3.0 + r2 * (0.4 + r2 * (2.0 / 7.0 + r2 * (2.0 / 9.0)))))
    return e.astype(jnp.float32) * LN2 + p


def _ln(y):
    return _lnraw(jnp.clip(y, 1e-12, 1.0))


def _sqrt(y):
    return jnp.exp(0.5 * _lnraw(jnp.maximum(y, 1e-36)))


def _chunk_params(tgt_v, off):
    i16 = lax.iota(jnp.int32, 16)
    tb = tgt_v[pl.ds(off, 16)].astype(jnp.int32)
    lab = tgt_v[pl.ds(256 + off, 16)].astype(jnp.int32)
    gx = tgt_v[pl.ds(512 + off, 16)] * float(G)
    gy = tgt_v[pl.ds(768 + off, 16)] * float(G)
    gw = tgt_v[pl.ds(1024 + off, 16)] * float(G)
    gh = tgt_v[pl.ds(1280 + off, 16)] * float(G)
    ious = []
    for a in range(NA):
        inter = jnp.minimum(SAW[a], gw) * jnp.minimum(SAH[a], gh)
        union = SAW[a] * SAH[a] + gw * gh - inter + 1e-16
        ious.append(inter / union)
    zero16 = jnp.zeros((16,), jnp.int32)
    best = zero16
    bv = ious[0]
    best = jnp.where(ious[1] > bv, 1, best)
    bv = jnp.maximum(bv, ious[1])
    best = jnp.where(ious[2] > bv, 2, best)
    gii = jnp.clip(gx.astype(jnp.int32), 0, G - 1)
    gji = jnp.clip(gy.astype(jnp.int32), 0, G - 1)
    cell = ((tb * NA + best) * G + gji) * G + gii
    del i16, zero16
    return dict(tb=tb, lab=lab, gx=gx, gy=gy, gw=gw, gh=gh, ious=ious,
                best=best, gii=gii, gji=gji, cell=cell)


def _body(pred_hbm, tgt_hbm, out_hbm, tgt_v, cells_v, rows_v,
          rb0_v, rb1_v, rb2_v, rb3_v, lbuf_v, part_v, sem3,
          rs0, rs1, rs2, rs3):
    i16 = lax.iota(jnp.int32, 16)
    f16 = i16.astype(jnp.float32)
    ones = jnp.ones((16,), jnp.float32)
    zeros = jnp.zeros((16,), jnp.float32)
    sid = lax.axis_index("s")
    cid = lax.axis_index("c")
    wid = sid * 2 + cid
    m = wid & 15

    pltpu.sync_copy(tgt_hbm, tgt_v)

    def p1(k, carry):
        prm = _chunk_params(tgt_v, k * 16)
        cells_v[pl.ds(k * 16, 16)] = prm["cell"]
        return carry

    lax.fori_loop(0, 16, p1, 0)

    prm = _chunk_params(tgt_v, m * 16)
    cell16 = prm["cell"]
    best = prm["best"]
    gx, gy, gw, gh = prm["gx"], prm["gy"], prm["gw"], prm["gh"]
    flrx = gx.astype(jnp.int32).astype(jnp.float32)
    flry = gy.astype(jnp.int32).astype(jnp.float32)
    tx16 = gx - flrx
    ty16 = (gy - flry + 0.5) * 0.5
    aw16 = jnp.where(best == 0, AW[0], jnp.where(best == 1, AW[1], AW[2]))
    ah16 = jnp.where(best == 0, AH[0], jnp.where(best == 1, AH[1], AH[2]))
    tw16 = _sqrt(gw / (aw16 / STRIDE)) * 0.5
    th16 = _sqrt(gh / (ah16 / STRIDE)) * 0.5
    gif = prm["gii"].astype(jnp.float32)
    gjf = prm["gji"].astype(jnp.float32)

    acc = jnp.zeros((16,), jnp.float32)

    @pl.when(wid < 16)
    def _obj():
        t16 = m * 16 + i16
        def dedup(tp, dup):
            cs = plsc.load_gather(cells_v, [jnp.broadcast_to(tp, (16,))])
            hit = (cell16 == cs) & (tp > t16)
            return jnp.where(hit, 1, dup)

        dup = lax.fori_loop(0, NT, dedup, jnp.zeros((16,), jnp.int32))
        wf = 1.0 - dup.astype(jnp.float32)

        for r in range(5):
            def fire(j, carry):
                k = r * 17 + j
                idx = cell16 * ROW + k
                pltpu.async_copy(pred_hbm.at[idx], rows_v.at[pl.ds(k * 16, 16)], sem3)
                return carry

            lax.fori_loop(0, 17, fire, 0)

            def drain(j, carry):
                k = r * 17 + j
                pltpu.make_async_copy(
                    pred_hbm.at[pl.ds(0, 16)], rows_v.at[pl.ds(k * 16, 16)], sem3
                ).wait()
                return carry

            lax.fori_loop(0, 17, drain, 0)

        idxl = cell16 * ROW + 5 + prm["lab"]
        pltpu.async_copy(pred_hbm.at[idxl], lbuf_v, sem3)
        pltpu.make_async_copy(pred_hbm.at[pl.ds(0, 16)], lbuf_v, sem3).wait()

        px = (rows_v[pl.ds(0, 16)] / STRIDE - gif + 0.5) * 0.5
        py = (rows_v[pl.ds(16, 16)] / STRIDE - gjf + 0.5) * 0.5
        pw = _sqrt(rows_v[pl.ds(32, 16)] / aw16) * 0.5
        ph = _sqrt(rows_v[pl.ds(48, 16)] / ah16) * 0.5
        conf = rows_v[pl.ds(64, 16)]
        lx = wf * (px - tx16) * (px - tx16)
        ly = wf * (py - ty16) * (py - ty16)
        lw = wf * (pw - tw16) * (pw - tw16)
        lh = wf * (ph - th16) * (ph - th16)
        cobj = wf * -_ln(conf)

        def clsbody(k, a):
            v = rows_v[pl.ds(k * 16, 16)]
            return a + -_ln(1.0 - v)

        scls = lax.fori_loop(5, ROW, clsbody, jnp.zeros((16,), jnp.float32))
        pl_ = lbuf_v[...]
        clstot = wf * scls + (-_ln(pl_) + _ln(1.0 - pl_))

        part = zeros
        part = jnp.where(f16 == 1.0, jnp.sum(lx), part)
        part = jnp.where(f16 == 2.0, jnp.sum(ly), part)
        part = jnp.where(f16 == 3.0, jnp.sum(lw), part)
        part = jnp.where(f16 == 4.0, jnp.sum(lh), part)
        part = jnp.where(f16 == 5.0, jnp.sum(cobj), part)
        part = jnp.where(f16 == 6.0, jnp.sum(clstot), part)
        part = jnp.where(f16 == 9.0, jnp.sum(wf), part)
        part_v[...] = part

    @pl.when(wid >= 16)
    def _noobj():
        zcells = []
        for a in range(NA):
            zc = ((prm["tb"] * NA + a) * G + prm["gji"]) * G + prm["gii"]
            zcells.append(zc)
            pltpu.async_copy(
                pred_hbm.at[zc * ROW + 4], rows_v.at[pl.ds(a * 16, 16)], sem3
            )
        zs = jnp.zeros((16,), jnp.float32)
        nz = jnp.zeros((16,), jnp.float32)
        for a in range(NA):
            pltpu.make_async_copy(
                pred_hbm.at[pl.ds(0, 16)], rows_v.at[pl.ds(a * 16, 16)], sem3
            ).wait()
            zb = ((best == a) | (prm["ious"][a] > 0.5)).astype(jnp.float32)
            zs = zs + zb * -_ln(1.0 - rows_v[pl.ds(a * 16, 16)])
            nz = nz + zb
        part = zeros
        part = jnp.where(f16 == 7.0, jnp.sum(zs), part)
        part = jnp.where(f16 == 8.0, jnp.sum(nz), part)
        part_v[...] = part

    rbufs = (rb0_v, rb1_v, rb2_v, rb3_v)
    rsems = (rs0, rs1, rs2, rs3)
    RLEN = [21760] * 15 + [18360]
    CELLS = [256] * 15 + [216]
    NJ = [16] * 15 + [14]
    ebase = wid * (4056 * ROW)

    def fire_read(d, b):
        n = RLEN[d]
        pltpu.async_copy(
            pred_hbm.at[pl.ds(ebase + d * 21760, n)], rbufs[b].at[pl.ds(0, n)],
            rsems[b])

    def wait_read(d, b):
        n = RLEN[d]
        pltpu.make_async_copy(
            pred_hbm.at[pl.ds(0, n)], rbufs[b].at[pl.ds(0, n)], rsems[b]
        ).wait()

    def chunk_sum(d, b, a0):
        full = d < 15

        def jbody(j, a):
            pos = j * 16 + i16
            idx = pos * ROW + 4
            if full:
                return a + -_ln(1.0 - plsc.load_gather(rbufs[b], [idx]))
            w = jnp.where(pos < CELLS[d], ones, zeros)
            idx = jnp.minimum(idx, RLEN[d] - 1)
            return a + w * -_ln(1.0 - plsc.load_gather(rbufs[b], [idx]))

        return lax.fori_loop(0, NJ[d], jbody, a0)

    acc16 = jnp.zeros((16,), jnp.float32)
    fire_read(0, 0)
    fire_read(1, 1)
    for d in range(16):
        b = d & 3
        wait_read(d, b)
        if d + 2 < 16:
            fire_read(d + 2, (d + 2) & 3)
        acc16 = chunk_sum(d, b, acc16)

    part_v[...] = part_v[...] + jnp.where(f16 == 0.0, jnp.sum(acc16), zeros)
    pltpu.sync_copy(part_v, out_hbm.at[wid])


@functools.partial(jax.jit, donate_argnums=())
def _sc_parts(pred_flat, tgt_flat):
    mesh = plsc.VectorSubcoreMesh(
        core_axis_name="c", subcore_axis_name="s", num_cores=2, num_subcores=16
    )
    return pl.kernel(
        _body,
        out_type=jax.ShapeDtypeStruct((32, 16), jnp.float32),
        mesh=mesh,
        compiler_params=pltpu.CompilerParams(needs_layout_passes=False),
        scratch_types=[
            pltpu.VMEM((6 * NT,), jnp.float32),
            pltpu.VMEM((NT,), jnp.int32),
            pltpu.VMEM((ROW * 16,), jnp.float32),
            pltpu.VMEM((21760,), jnp.float32),
            pltpu.VMEM((21760,), jnp.float32),
            pltpu.VMEM((21760,), jnp.float32),
            pltpu.VMEM((21760,), jnp.float32),
            pltpu.VMEM((16,), jnp.float32),
            pltpu.VMEM((16,), jnp.float32),
            pltpu.SemaphoreType.DMA,
            pltpu.SemaphoreType.DMA,
            pltpu.SemaphoreType.DMA,
            pltpu.SemaphoreType.DMA,
            pltpu.SemaphoreType.DMA,
        ],
    )(pred_flat, tgt_flat)


def kernel(pred, targets):
    pred_flat = pred.reshape(-1)
    tgt_flat = jnp.transpose(targets).reshape(-1)
    parts = _sc_parts(pred_flat, tgt_flat)
    s = jnp.sum(parts, axis=0)
    dense, lx, ly, lw, lh, cobj, cls_ = s[0], s[1], s[2], s[3], s[4], s[5], s[6]
    zsum, nzero, nobj = s[7], s[8], s[9]
    total = (
        10.0 * (lx + ly + lw + lh) / nobj
        + cobj / nobj
        + 100.0 * (dense - zsum) / (float(NCELL) - nzero)
        + 10.0 * cls_ / (nobj * float(C))
    )
    return (pred, total.astype(jnp.float32))


# speedup vs baseline: 6.4953x; 5.2156x over previous
import functools

import jax
import jax.numpy as jnp
from jax import lax
from jax.experimental import pallas as pl
from jax.experimental.pallas import tpu as pltpu
from jax.experimental.pallas import tpu_sc as plsc

B = 16
NA = 3
G = 52
C = 80
ROW = C + 5
NG = NA * G * G
NCELL = B * NG
NT = 256
STRIDE = 8.0
LN2 = 0.6931471805599453
AW = (10.0, 16.0, 33.0)
AH = (13.0, 30.0, 23.0)
SAW = tuple(a / STRIDE for a in AW)
SAH = tuple(a / STRIDE for a in AH)
HALF = NG // 2


def _lnraw(y):
    bits = lax.bitcast_convert_type(y, jnp.int32)
    e = (bits >> 23) - 127
    m = lax.bitcast_convert_type((bits & 0x007FFFFF) | 0x3F800000, jnp.float32)
    big = m > 1.4142135623730951
    m = jnp.where(big, m * 0.5, m)
    e = jnp.where(big, e + 1, e)
    r = (m - 1.0) / (m + 1.0)
    r2 = r * r
    p = r * (2.0 + r2 * (2.0 / 3.0 + r2 * (0.4 + r2 * (2.0 / 7.0 + r2 * (2.0 / 9.0)))))
    return e.astype(jnp.float32) * LN2 + p


def _ln(y):
    return _lnraw(jnp.clip(y, 1e-12, 1.0))


def _sqrt(y):
    return jnp.exp(0.5 * _lnraw(jnp.maximum(y, 1e-36)))


def _chunk_params(tgt_v, off):
    tb = tgt_v[pl.ds(off, 16)].astype(jnp.int32)
    lab = tgt_v[pl.ds(256 + off, 16)].astype(jnp.int32)
    gx = tgt_v[pl.ds(512 + off, 16)] * float(G)
    gy = tgt_v[pl.ds(768 + off, 16)] * float(G)
    gw = tgt_v[pl.ds(1024 + off, 16)] * float(G)
    gh = tgt_v[pl.ds(1280 + off, 16)] * float(G)
    ious = []
    for a in range(NA):
        inter = jnp.minimum(SAW[a], gw) * jnp.minimum(SAH[a], gh)
        union = SAW[a] * SAH[a] + gw * gh - inter + 1e-16
        ious.append(inter / union)
    best = jnp.zeros((16,), jnp.int32)
    bv = ious[0]
    best = jnp.where(ious[1] > bv, 1, best)
    bv = jnp.maximum(bv, ious[1])
    best = jnp.where(ious[2] > bv, 2, best)
    gii = jnp.clip(gx.astype(jnp.int32), 0, G - 1)
    gji = jnp.clip(gy.astype(jnp.int32), 0, G - 1)
    nloc = (best * G + gji) * G + gii
    cell = tb * NG + nloc
    return dict(tb=tb, lab=lab, gx=gx, gy=gy, gw=gw, gh=gh, ious=ious,
                best=best, gii=gii, gji=gji, nloc=nloc, cell=cell)


def _rowpos(fv):
    ch = fv // NG
    m = fv - ch * NG
    return ch, m


def _body(pred_hbm, tgt_hbm, out_hbm, tgt_v, cells_v, rows_v, wa_v, wb_v,
          rb0_v, rb1_v, rb2_v, rb3_v, part_v, sem3,
          rs0, rs1, rs2, rs3):
    i16 = lax.iota(jnp.int32, 16)
    f16 = i16.astype(jnp.float32)
    ones = jnp.ones((16,), jnp.float32)
    zeros = jnp.zeros((16,), jnp.float32)
    sid = lax.axis_index("s")
    cid = lax.axis_index("c")
    wid = sid * 2 + cid
    m_chunk = wid & 15

    pltpu.sync_copy(tgt_hbm, tgt_v)

    def p1(k, carry):
        cells_v[pl.ds(k * 16, 16)] = _chunk_params(tgt_v, k * 16)["cell"]
        return carry

    lax.fori_loop(0, 16, p1, 0)

    prm = _chunk_params(tgt_v, m_chunk * 16)
    cell16 = prm["cell"]
    best = prm["best"]
    tb16 = prm["tb"]
    gx, gy, gw, gh = prm["gx"], prm["gy"], prm["gw"], prm["gh"]
    flrx = gx.astype(jnp.int32).astype(jnp.float32)
    flry = gy.astype(jnp.int32).astype(jnp.float32)
    tx16 = gx - flrx
    ty16 = (gy - flry + 0.5) * 0.5
    aw16 = jnp.where(best == 0, AW[0], jnp.where(best == 1, AW[1], AW[2]))
    ah16 = jnp.where(best == 0, AH[0], jnp.where(best == 1, AH[1], AH[2]))
    tw16 = _sqrt(gw / (aw16 / STRIDE)) * 0.5
    th16 = _sqrt(gh / (ah16 / STRIDE)) * 0.5
    gif = prm["gii"].astype(jnp.float32)
    gjf = prm["gji"].astype(jnp.float32)

    @pl.when(wid < 16)
    def _obj():
        t16 = m_chunk * 16 + i16

        def dedup(tp, dup):
            cs = plsc.load_gather(cells_v, [jnp.broadcast_to(tp, (16,))])
            hit = (cell16 == cs) & (tp > t16)
            return jnp.where(hit, 1, dup)

        dup = lax.fori_loop(0, NT, dedup, jnp.zeros((16,), jnp.int32))
        wf = 1.0 - dup.astype(jnp.float32)

        fo = prm["nloc"] * ROW
        ch16, m016 = _rowpos(fo)
        rta16 = (ch16 >> 3) << 3
        ca16 = jnp.minimum((m016 >> 7) << 7, NG + 80 - 256)
        ch2_16 = jnp.minimum(ch16 + 1, ROW - 1)
        rtb16 = (ch2_16 >> 3) << 3

        def fire_t(t, buf):
            bt = tb16[t]
            pltpu.async_copy(
                pred_hbm.at[bt, pl.ds(pl.multiple_of(rta16[t], 8), 8), pl.ds(pl.multiple_of(ca16[t], 128), 256)],
                buf.at[0], sem3)
            pltpu.async_copy(
                pred_hbm.at[bt, pl.ds(pl.multiple_of(rtb16[t], 8), 8), pl.ds(0, 256)],
                buf.at[1], sem3)

        def wait_t(buf):
            for _ in range(2):
                pltpu.make_async_copy(
                    pred_hbm.at[0, pl.ds(0, 8), pl.ds(0, 256)], buf.at[0], sem3
                ).wait()

        fire_t(0, wa_v)
        fire_t(1, wb_v)
        for t in range(16):
            buf = wa_v if t % 2 == 0 else wb_v
            wait_t(buf)
            if t + 2 < 16:
                fire_t(t + 2, buf)
            rowA = jnp.broadcast_to(ch16[t] - rta16[t], (16,))
            rowB = jnp.broadcast_to(ch2_16[t] - rtb16[t], (16,))
            m0t = jnp.broadcast_to(m016[t], (16,))
            cat = jnp.broadcast_to(ca16[t], (16,))
            for j in range(6):
                k16 = j * 16 + i16
                inA = (m0t + k16) < NG
                sel = jnp.where(inA, 0, 1)
                row = jnp.where(inA, rowA, rowB)
                lane = jnp.where(inA, m0t - cat + k16, m0t + k16 - NG)
                lane = jnp.clip(lane, 0, 255)
                g = plsc.load_gather(buf, [sel, row, lane])
                rows_v[pl.ds(t * 96 + j * 16, 16)] = g

        def chan(k):
            return plsc.load_gather(rows_v, [i16 * 96 + k])

        px = (chan(0) / STRIDE - gif + 0.5) * 0.5
        py = (chan(1) / STRIDE - gjf + 0.5) * 0.5
        pw = _sqrt(chan(2) / aw16) * 0.5
        ph = _sqrt(chan(3) / ah16) * 0.5
        conf = chan(4)
        lx = wf * (px - tx16) * (px - tx16)
        ly = wf * (py - ty16) * (py - ty16)
        lw = wf * (pw - tw16) * (pw - tw16)
        lh = wf * (ph - th16) * (ph - th16)
        cobj = wf * -_ln(conf)

        def clsbody(k, a):
            return a + -_ln(1.0 - chan(k))

        scls = lax.fori_loop(5, ROW, clsbody, jnp.zeros((16,), jnp.float32))
        pl_ = chan(5 + prm["lab"])
        clstot = wf * scls + (-_ln(pl_) + _ln(1.0 - pl_))

        part = zeros
        part = jnp.where(f16 == 1.0, jnp.sum(lx), part)
        part = jnp.where(f16 == 2.0, jnp.sum(ly), part)
        part = jnp.where(f16 == 3.0, jnp.sum(lw), part)
        part = jnp.where(f16 == 4.0, jnp.sum(lh), part)
        part = jnp.where(f16 == 5.0, jnp.sum(cobj), part)
        part = jnp.where(f16 == 6.0, jnp.sum(clstot), part)
        part = jnp.where(f16 == 9.0, jnp.sum(wf), part)
        part_v[...] = part

    @pl.when(wid >= 16)
    def _noobj():
        chv, mv, rtv, cbv, zbv = [], [], [], [], []
        for a in range(NA):
            nl = (a * G + prm["gji"]) * G + prm["gii"]
            fz = nl * ROW + 4
            ch, mm = _rowpos(fz)
            chv.append(ch)
            mv.append(mm)
            rtv.append((ch >> 3) << 3)
            cbv.append((mm >> 7) << 7)
            zbv.append(((best == a) | (prm["ious"][a] > 0.5)).astype(jnp.float32))
        zslots = (wa_v, wb_v)

        def fire_q(q):
            a, t = q // 16, q % 16
            s = q % 4
            zslots[s // 2].at[s % 2]
            pltpu.async_copy(
                pred_hbm.at[tb16[t], pl.ds(pl.multiple_of(rtv[a][t], 8), 8), pl.ds(pl.multiple_of(cbv[a][t], 128), 128)],
                zslots[s // 2].at[s % 2, :, pl.ds(0, 128)], sem3)

        def wait_q(q):
            s = q % 4
            pltpu.make_async_copy(
                pred_hbm.at[0, pl.ds(0, 8), pl.ds(0, 128)],
                zslots[s // 2].at[s % 2, :, pl.ds(0, 128)], sem3).wait()

        for q in range(4):
            fire_q(q)
        for q in range(48):
            a, t = q // 16, q % 16
            s = q % 4
            wait_q(q)
            row = jnp.broadcast_to(chv[a][t] - rtv[a][t], (16,))
            lane = jnp.broadcast_to(mv[a][t] - cbv[a][t], (16,))
            g = plsc.load_gather(zslots[s // 2], [jnp.broadcast_to(s % 2, (16,)), row, lane])
            plsc.store_scatter(rows_v, [jnp.broadcast_to(q, (16,))], g)
            if q + 4 < 48:
                fire_q(q + 4)
        zs = jnp.zeros((16,), jnp.float32)
        nz = jnp.zeros((16,), jnp.float32)
        for a in range(NA):
            confz = rows_v[pl.ds(a * 16, 16)]
            zs = zs + zbv[a] * -_ln(1.0 - confz)
            nz = nz + zbv[a]
        part = zeros
        part = jnp.where(f16 == 7.0, jnp.sum(zs), part)
        part = jnp.where(f16 == 8.0, jnp.sum(nz), part)
        part_v[...] = part

    rbufs = (rb0_v, rb1_v, rb2_v, rb3_v)
    rsems = (rs0, rs1, rs2, rs3)

    def unit_coords(u):
        b = u // 44
        pos = u - b * 44
        rb = pos >> 2
        cc = pos & 3
        return b, rb, cc

    def fire(d, s):
        b, rb, cc = unit_coords(wid * 22 + d)
        pltpu.async_copy(
            pred_hbm.at[b, pl.ds(pl.multiple_of(rb * 8, 8), 8), pl.ds(pl.multiple_of(cc * 2048, 128), 2048)],
            rbufs[s], rsems[s])

    def wait_unit(s):
        pltpu.make_async_copy(
            pred_hbm.at[0, pl.ds(0, 8), pl.ds(0, 2048)], rbufs[s], rsems[s]
        ).wait()

    def unit_sum(d, s, a0):
        _, rb, cc = unit_coords(wid * 22 + d)

        def rbody(rr, a):
            ch = rb * 8 + rr
            w0 = ch * NG + cc * 2048
            rowok = jnp.broadcast_to(ch, (16,)) < ROW
            n0 = ((w0 + 80).astype(jnp.float32) * (1.0 / 85.0) + 0.004
                  ).astype(jnp.int32)

            def jbody(j, aa):
                n16 = n0 + j * 16 + i16
                fb = n16 * ROW + 4
                local = fb - w0
                laneok = (fb - ch * NG) < NG
                ok = (local >= 0) & (local < 2048) & laneok & rowok
                w = jnp.where(ok, ones, zeros)
                idx = jnp.clip(local, 0, 2047)
                rr16 = jnp.broadcast_to(rr, (16,))
                return aa + w * -_ln(1.0 - plsc.load_gather(rbufs[s], [rr16, idx]))

            return lax.fori_loop(0, 2, jbody, a)

        return lax.fori_loop(0, 8, rbody, a0)

    acc16 = jnp.zeros((16,), jnp.float32)
    fire(0, 0)
    fire(1, 1)
    for d in range(22):
        s = d & 3
        wait_unit(s)
        if d + 2 < 22:
            fire(d + 2, (d + 2) & 3)
        acc16 = unit_sum(d, s, acc16)

    part_v[...] = part_v[...] + jnp.where(f16 == 0.0, jnp.sum(acc16), zeros)
    pltpu.sync_copy(part_v, out_hbm.at[wid])


@functools.partial(jax.jit, donate_argnums=())
def _sc_parts(pred, tgt_flat):
    mesh = plsc.VectorSubcoreMesh(
        core_axis_name="c", subcore_axis_name="s", num_cores=2, num_subcores=16
    )
    return pl.kernel(
        _body,
        out_type=jax.ShapeDtypeStruct((32, 16), jnp.float32),
        mesh=mesh,
        compiler_params=pltpu.CompilerParams(
            needs_layout_passes=False, disable_bounds_checks=True
        ),
        scratch_types=[
            pltpu.VMEM((6 * NT,), jnp.float32),
            pltpu.VMEM((NT,), jnp.int32),
            pltpu.VMEM((16 * 96,), jnp.float32),
            pltpu.VMEM((2, 8, 256), jnp.float32),
            pltpu.VMEM((2, 8, 256), jnp.float32),
            pltpu.VMEM((8, 2048), jnp.float32),
            pltpu.VMEM((8, 2048), jnp.float32),
            pltpu.VMEM((8, 2048), jnp.float32),
            pltpu.VMEM((8, 2048), jnp.float32),
            pltpu.VMEM((16,), jnp.float32),
            pltpu.SemaphoreType.DMA,
            pltpu.SemaphoreType.DMA,
            pltpu.SemaphoreType.DMA,
            pltpu.SemaphoreType.DMA,
            pltpu.SemaphoreType.DMA,
        ],
    )(pred, tgt_flat)


def kernel(pred, targets):
    tgt_flat = jnp.transpose(targets).reshape(-1)
    parts = _sc_parts(pred, tgt_flat)
    s = jnp.sum(parts, axis=0)
    dense, lx, ly, lw, lh, cobj, cls_ = s[0], s[1], s[2], s[3], s[4], s[5], s[6]
    zsum, nzero, nobj = s[7], s[8], s[9]
    total = (
        10.0 * (lx + ly + lw + lh) / nobj
        + cobj / nobj
        + 100.0 * (dense - zsum) / (float(NCELL) - nzero)
        + 10.0 * cls_ / (nobj * float(C))
    )
    return (pred, total.astype(jnp.float32))
